# baseline (speedup 1.0000x reference)
import functools

import jax
import jax.numpy as jnp
from jax import lax
from jax.experimental import pallas as pl
from jax.experimental.pallas import tpu as pltpu
from jax.experimental.pallas import tpu_sc as plsc

BATCH = 64
T = 1024
D = 256
K = 1024
EPS = 1e-5
F32 = jnp.float32
BF16 = jnp.bfloat16
CHG = 256


def _conv1d(x, w, b, pad):
    out = jax.lax.conv_general_dilated(
        x, w, window_strides=(1,), padding=[(pad, pad)],
        dimension_numbers=('NCH', 'OIH', 'NCH'))
    return out + b[None, :, None]


def _batchnorm(x, gamma, beta):
    mean = jnp.mean(x, axis=(0, 2), keepdims=True)
    var = jnp.var(x, axis=(0, 2), keepdims=True)
    xn = (x - mean) / jnp.sqrt(var + EPS)
    return xn * gamma[None, :, None] + beta[None, :, None]


def _vq_body(ze_in_ref, cb_ref, cb2_ref, ids_ref):
    ze = ze_in_ref[0]
    cross = jax.lax.dot_general(ze.astype(BF16), cb_ref[...].astype(BF16),
                                (((1,), (1,)), ((), ())),
                                preferred_element_type=F32)
    scores = cb2_ref[...] - 2.0 * cross
    m = jnp.min(scores, axis=1, keepdims=True)
    colk = jax.lax.broadcasted_iota(jnp.int32, (T, K), 1)
    ids_ref[0, 0] = jnp.min(jnp.where(scores <= m, colk, K), axis=1)


def _sc_gather(table, idx):
    n = idx.shape[0]
    info = plsc.get_sparse_core_info()
    nw = info.num_cores * info.num_subcores
    bpw = n // nw
    mesh = plsc.VectorSubcoreMesh(core_axis_name="c", subcore_axis_name="s")

    @functools.partial(
        pl.kernel, mesh=mesh,
        out_type=jax.ShapeDtypeStruct((n, D), F32),
        scratch_types=[
            pltpu.VMEM((CHG,), jnp.int32),
            pltpu.VMEM((CHG, D), F32),
            pltpu.SemaphoreType.DMA,
        ],
    )
    def k(table_hbm, idx_hbm, out_hbm, idx_v, rows_v, sem):
        wid = lax.axis_index("s") * info.num_cores + lax.axis_index("c")
        base = wid * bpw
        for j in range(bpw // CHG):
            off = base + j * CHG
            pltpu.sync_copy(idx_hbm.at[pl.ds(off, CHG)], idx_v)
            pltpu.async_copy(table_hbm.at[idx_v], rows_v, sem).wait()
            pltpu.sync_copy(rows_v, out_hbm.at[pl.ds(off, CHG)])

    return k(table, idx)


def kernel(x, w1, b1, g1, be1, w2, b2, g2, be2, w3, b3, g3, be3, w4, b4,
           codebook):
    h = jnp.swapaxes(x, -1, -2)
    h = jax.nn.relu(_batchnorm(_conv1d(h, w1, b1, 2), g1, be1))
    h = jax.nn.relu(_batchnorm(_conv1d(h, w2, b2, 1), g2, be2))
    h = jax.nn.relu(_batchnorm(_conv1d(h, w3, b3, 2), g3, be3))
    z_e = jnp.swapaxes(_conv1d(h, w4, b4, 1), -1, -2)

    cb2 = jnp.sum(codebook * codebook, axis=-1).reshape(1, K)
    ids = pl.pallas_call(
        _vq_body,
        grid=(BATCH,),
        in_specs=[pl.BlockSpec((1, T, D), lambda i: (i, 0, 0)),
                  pl.BlockSpec((K, D), lambda i: (0, 0)),
                  pl.BlockSpec((1, K), lambda i: (0, 0))],
        out_specs=pl.BlockSpec((1, 1, T), lambda i: (i, 0, 0)),
        out_shape=jax.ShapeDtypeStruct((BATCH, 1, T), jnp.int32),
    )(z_e, codebook, cb2)

    rows = _sc_gather(codebook, ids.reshape(BATCH * T))
    zq = rows.reshape(BATCH, T, D)
    zq = z_e + (zq - z_e)
    return z_e, zq


# speedup vs baseline: 1.5798x; 1.5798x over previous
import jax
import jax.numpy as jnp
from jax.experimental import pallas as pl

BATCH = 64
T = 1024
D = 256
K = 1024
EPS = 1e-5
F32 = jnp.float32
BF16 = jnp.bfloat16


def _conv1d(x, w, b, pad):
    out = jax.lax.conv_general_dilated(
        x, w, window_strides=(1,), padding=[(pad, pad)],
        dimension_numbers=('NCH', 'OIH', 'NCH'))
    return out + b[None, :, None]


def _batchnorm(x, gamma, beta):
    mean = jnp.mean(x, axis=(0, 2), keepdims=True)
    var = jnp.var(x, axis=(0, 2), keepdims=True)
    xn = (x - mean) / jnp.sqrt(var + EPS)
    return xn * gamma[None, :, None] + beta[None, :, None]


def _vq_body(ze_in_ref, cb_ref, cb2_ref, zq_ref):
    ze = ze_in_ref[0]
    cb = cb_ref[...]
    cross = jax.lax.dot_general(ze.astype(BF16), cb.astype(BF16),
                                (((1,), (1,)), ((), ())),
                                preferred_element_type=F32)
    scores = cb2_ref[...] - 2.0 * cross
    m = jnp.min(scores, axis=1, keepdims=True)
    colk = jax.lax.broadcasted_iota(jnp.int32, (T, K), 1)
    ids = jnp.min(jnp.where(scores <= m, colk, K), axis=1)
    onehot = (colk == ids[:, None]).astype(F32)
    zq = jax.lax.dot_general(onehot, cb, (((1,), (0,)), ((), ())),
                             preferred_element_type=F32,
                             precision=jax.lax.Precision.HIGHEST)
    zq_ref[0] = ze + (zq - ze)


def kernel(x, w1, b1, g1, be1, w2, b2, g2, be2, w3, b3, g3, be3, w4, b4,
           codebook):
    h = jnp.swapaxes(x, -1, -2)
    h = jax.nn.relu(_batchnorm(_conv1d(h, w1, b1, 2), g1, be1))
    h = jax.nn.relu(_batchnorm(_conv1d(h, w2, b2, 1), g2, be2))
    h = jax.nn.relu(_batchnorm(_conv1d(h, w3, b3, 2), g3, be3))
    z_e = jnp.swapaxes(_conv1d(h, w4, b4, 1), -1, -2)

    cb2 = jnp.sum(codebook * codebook, axis=-1).reshape(1, K)
    zq = pl.pallas_call(
        _vq_body,
        grid=(BATCH,),
        in_specs=[pl.BlockSpec((1, T, D), lambda i: (i, 0, 0)),
                  pl.BlockSpec((K, D), lambda i: (0, 0)),
                  pl.BlockSpec((1, K), lambda i: (0, 0))],
        out_specs=pl.BlockSpec((1, T, D), lambda i: (i, 0, 0)),
        out_shape=jax.ShapeDtypeStruct((BATCH, T, D), F32),
    )(z_e, codebook, cb2)
    return z_e, zq
